# Initial kernel scaffold; baseline (speedup 1.0000x reference)
import jax
import jax.numpy as jnp
from jax.experimental import pallas as pl


def kernel(x, edge_index, W1, b1, W2, b2):
    raise NotImplementedError("write your pallas kernel here")


# speedup vs baseline: 8.1289x; 8.1289x over previous
import functools

import jax
import jax.numpy as jnp
from jax import lax
from jax.experimental import pallas as pl
from jax.experimental.pallas import tpu as pltpu
from jax.experimental.pallas import tpu_sc as plsc

N = 10000
D = 128
E = 320000

NC = 2
NS = 16
CH = 128
N_PAD = 10112
E_PAD = 327680
CHUNKS_PER_TILE = E_PAD // (NC * NS * CH)
ROWS_PER_TILE = N_PAD // NS
G = 40
ZROWS = 8

_MESH = plsc.VectorSubcoreMesh(
    core_axis_name="c", subcore_axis_name="s", num_cores=NC, num_subcores=NS)


def _sc_deg_body(dst2d, out, ones_v, zero_v, idx_v, deg_sh):
    c = lax.axis_index("c")
    s = lax.axis_index("s")
    for i in range(CH):
        ones_v[i, :] = jnp.ones((16,), jnp.float32)
    for i in range(ZROWS):
        zero_v[i, :] = jnp.zeros((16,), jnp.float32)

    @pl.loop(0, ROWS_PER_TILE // ZROWS)
    def _z(r):
        pltpu.sync_copy(zero_v, deg_sh.at[pl.ds(s * ROWS_PER_TILE + r * ZROWS, ZROWS)])
    plsc.subcore_barrier()
    row_base = (c * NS + s) * CHUNKS_PER_TILE
    pltpu.sync_copy(dst2d.at[pl.ds(row_base, CHUNKS_PER_TILE)], idx_v)

    @pl.loop(0, CHUNKS_PER_TILE)
    def _chunk(j):
        pltpu.sync_copy(ones_v, deg_sh.at[idx_v.at[j]], add=True)

    plsc.subcore_barrier()
    pltpu.sync_copy(deg_sh.at[pl.ds(s * ROWS_PER_TILE, ROWS_PER_TILE)],
                    out.at[c, pl.ds(s * ROWS_PER_TILE, ROWS_PER_TILE)])


_sc_deg = pl.kernel(
    _sc_deg_body,
    out_type=jax.ShapeDtypeStruct((NC, N_PAD, 16), jnp.float32),
    mesh=_MESH,
    scratch_types=[
        pltpu.VMEM((CH, 16), jnp.float32),
        pltpu.VMEM((ZROWS, 16), jnp.float32),
        pltpu.VMEM((CHUNKS_PER_TILE, CH), jnp.int32),
        pltpu.VMEM_SHARED((N_PAD, 16), jnp.float32),
    ],
)


def _sc_agg_body(hs, src2d, dst2d, out, srcv, dstv, rows_a, rows_b, zero_v,
                 sem_a, sem_b, acc_sh):
    c = lax.axis_index("c")
    s = lax.axis_index("s")
    for i in range(ZROWS):
        for g in range(D // 16):
            zero_v[i, pl.ds(g * 16, 16)] = jnp.zeros((16,), jnp.float32)

    @pl.loop(0, ROWS_PER_TILE // ZROWS)
    def _z(r):
        pltpu.sync_copy(zero_v, acc_sh.at[pl.ds(s * ROWS_PER_TILE + r * ZROWS, ZROWS)])

    plsc.subcore_barrier()
    row_base = (c * NS + s) * CHUNKS_PER_TILE

    for grp in range(CHUNKS_PER_TILE // G):
        pltpu.sync_copy(src2d.at[pl.ds(row_base + grp * G, G)], srcv)
        pltpu.sync_copy(dst2d.at[pl.ds(row_base + grp * G, G)], dstv)
        pltpu.async_copy(hs.at[srcv.at[0]], rows_a, sem_a)

        @pl.loop(0, G, step=2)
        def _pair(j):
            pltpu.make_async_copy(hs.at[srcv.at[j]], rows_a, sem_a).wait()
            pltpu.async_copy(hs.at[srcv.at[j + 1]], rows_b, sem_b)
            pltpu.sync_copy(rows_a, acc_sh.at[dstv.at[j]], add=True)
            pltpu.make_async_copy(hs.at[srcv.at[j + 1]], rows_b, sem_b).wait()
            pltpu.async_copy(hs.at[srcv.at[jnp.minimum(j + 2, G - 1)]], rows_a, sem_a)
            pltpu.sync_copy(rows_b, acc_sh.at[dstv.at[j + 1]], add=True)

        pltpu.make_async_copy(hs.at[srcv.at[G - 1]], rows_a, sem_a).wait()

    plsc.subcore_barrier()
    pltpu.sync_copy(acc_sh.at[pl.ds(s * ROWS_PER_TILE, ROWS_PER_TILE)],
                    out.at[c, pl.ds(s * ROWS_PER_TILE, ROWS_PER_TILE)])


_sc_agg = pl.kernel(
    _sc_agg_body,
    out_type=jax.ShapeDtypeStruct((NC, N_PAD, D), jnp.float32),
    mesh=_MESH,
    scratch_types=[
        pltpu.VMEM((G, CH), jnp.int32),
        pltpu.VMEM((G, CH), jnp.int32),
        pltpu.VMEM((CH, D), jnp.float32),
        pltpu.VMEM((CH, D), jnp.float32),
        pltpu.VMEM((ZROWS, D), jnp.float32),
        pltpu.SemaphoreType.DMA,
        pltpu.SemaphoreType.DMA,
        pltpu.VMEM_SHARED((N_PAD, D), jnp.float32),
    ],
)


RB = 400
GRID = N // RB


def _tc1_body(deg_ref, x_ref, w_ref, hs_ref, dis_ref):
    dp = deg_ref[...]
    deg = 1.0 + jnp.sum(dp[0] + dp[1], axis=-1, keepdims=True) * (1.0 / 16.0)
    dis = lax.rsqrt(deg)
    h = jnp.dot(x_ref[...], w_ref[...], preferred_element_type=jnp.float32)
    hs_ref[...] = dis * h
    dis_ref[...] = dis


def _tc2_body(parts_ref, hs_ref, dis_ref, b_ref, w_ref, out_ref):
    p = parts_ref[...]
    dis = dis_ref[...]
    z = dis * (p[0] + p[1] + hs_ref[...]) + b_ref[...]
    h = jnp.maximum(z, 0.0)
    out_ref[...] = dis * jnp.dot(h, w_ref[...], preferred_element_type=jnp.float32)


def _tc3_body(parts_ref, hs_ref, dis_ref, b_ref, out_ref):
    p = parts_ref[...]
    out_ref[...] = dis_ref[...] * (p[0] + p[1] + hs_ref[...]) + b_ref[...]


_tc1 = pl.pallas_call(
    _tc1_body,
    grid=(GRID,),
    in_specs=[
        pl.BlockSpec((NC, RB, 16), lambda i: (0, i, 0)),
        pl.BlockSpec((RB, D), lambda i: (i, 0)),
        pl.BlockSpec((D, D), lambda i: (0, 0)),
    ],
    out_specs=[
        pl.BlockSpec((RB, D), lambda i: (i, 0)),
        pl.BlockSpec((RB, 1), lambda i: (i, 0)),
    ],
    out_shape=[
        jax.ShapeDtypeStruct((N, D), jnp.float32),
        jax.ShapeDtypeStruct((N, 1), jnp.float32),
    ],
)

_tc2 = pl.pallas_call(
    _tc2_body,
    grid=(GRID,),
    in_specs=[
        pl.BlockSpec((NC, RB, D), lambda i: (0, i, 0)),
        pl.BlockSpec((RB, D), lambda i: (i, 0)),
        pl.BlockSpec((RB, 1), lambda i: (i, 0)),
        pl.BlockSpec((1, D), lambda i: (0, 0)),
        pl.BlockSpec((D, D), lambda i: (0, 0)),
    ],
    out_specs=pl.BlockSpec((RB, D), lambda i: (i, 0)),
    out_shape=jax.ShapeDtypeStruct((N, D), jnp.float32),
)

_tc3 = pl.pallas_call(
    _tc3_body,
    grid=(GRID,),
    in_specs=[
        pl.BlockSpec((NC, RB, D), lambda i: (0, i, 0)),
        pl.BlockSpec((RB, D), lambda i: (i, 0)),
        pl.BlockSpec((RB, 1), lambda i: (i, 0)),
        pl.BlockSpec((1, D), lambda i: (0, 0)),
    ],
    out_specs=pl.BlockSpec((RB, D), lambda i: (i, 0)),
    out_shape=jax.ShapeDtypeStruct((N, D), jnp.float32),
)


def kernel(x, edge_index, W1, b1, W2, b2):
    npad = E_PAD - E
    pad_dst = N + (jnp.arange(npad, dtype=jnp.int32) % (N_PAD - N))
    src2d = jnp.concatenate(
        [edge_index[0], jnp.zeros((npad,), jnp.int32)]).reshape(E_PAD // CH, CH)
    dst2d = jnp.concatenate([edge_index[1], pad_dst]).reshape(E_PAD // CH, CH)

    deg_parts = _sc_deg(dst2d)
    hs1, dis = _tc1(deg_parts, x, W1)
    parts1 = _sc_agg(hs1, src2d, dst2d)
    hs2 = _tc2(parts1, hs1, dis, b1.reshape(1, D), W2)
    parts2 = _sc_agg(hs2, src2d, dst2d)
    return _tc3(parts2, hs2, dis, b2.reshape(1, D))


# speedup vs baseline: 25.6536x; 3.1558x over previous
import functools

import jax
import jax.numpy as jnp
from jax import lax
from jax.experimental import pallas as pl
from jax.experimental.pallas import tpu as pltpu
from jax.experimental.pallas import tpu_sc as plsc

N = 10000
D = 128
E = 320000

NC = 2
NS = 16
CH = 128
N_PAD = 10112
E_PAD = 327680
CHUNKS_PER_TILE = E_PAD // (NC * NS * CH)
ROWS_PER_TILE = N_PAD // NS
G = 40
ZROWS = 8

_MESH = plsc.VectorSubcoreMesh(
    core_axis_name="c", subcore_axis_name="s", num_cores=NC, num_subcores=NS)


def _sc_deg_body(dst2d, out, ones_v, zero_v, idx_v, deg_sh):
    c = lax.axis_index("c")
    s = lax.axis_index("s")
    for i in range(CH):
        ones_v[i, :] = jnp.ones((16,), jnp.float32)
    for i in range(ZROWS):
        zero_v[i, :] = jnp.zeros((16,), jnp.float32)

    @pl.loop(0, ROWS_PER_TILE // ZROWS)
    def _z(r):
        pltpu.sync_copy(zero_v, deg_sh.at[pl.ds(s * ROWS_PER_TILE + r * ZROWS, ZROWS)])
    plsc.subcore_barrier()
    row_base = (c * NS + s) * CHUNKS_PER_TILE
    pltpu.sync_copy(dst2d.at[pl.ds(row_base, CHUNKS_PER_TILE)], idx_v)

    @pl.loop(0, CHUNKS_PER_TILE)
    def _chunk(j):
        pltpu.sync_copy(ones_v, deg_sh.at[idx_v.at[j]], add=True)

    plsc.subcore_barrier()
    pltpu.sync_copy(deg_sh.at[pl.ds(s * ROWS_PER_TILE, ROWS_PER_TILE)],
                    out.at[c, pl.ds(s * ROWS_PER_TILE, ROWS_PER_TILE)])


_sc_deg = pl.kernel(
    _sc_deg_body,
    out_type=jax.ShapeDtypeStruct((NC, N_PAD, 16), jnp.float32),
    mesh=_MESH,
    scratch_types=[
        pltpu.VMEM((CH, 16), jnp.float32),
        pltpu.VMEM((ZROWS, 16), jnp.float32),
        pltpu.VMEM((CHUNKS_PER_TILE, CH), jnp.int32),
        pltpu.VMEM_SHARED((N_PAD, 16), jnp.float32),
    ],
)


def _sc_agg_body(hs, src2d, dst2d, out, srcv, dstv, rows_a, rows_b, zero_v,
                 sem_a, sem_b, acc_sh):
    c = lax.axis_index("c")
    s = lax.axis_index("s")
    for i in range(ZROWS):
        for g in range(D // 16):
            zero_v[i, pl.ds(g * 16, 16)] = jnp.zeros((16,), jnp.float32)

    @pl.loop(0, ROWS_PER_TILE // ZROWS)
    def _z(r):
        pltpu.sync_copy(zero_v, acc_sh.at[pl.ds(s * ROWS_PER_TILE + r * ZROWS, ZROWS)])

    plsc.subcore_barrier()
    row_base = (c * NS + s) * CHUNKS_PER_TILE

    for grp in range(CHUNKS_PER_TILE // G):
        pltpu.sync_copy(src2d.at[pl.ds(row_base + grp * G, G)], srcv)
        pltpu.sync_copy(dst2d.at[pl.ds(row_base + grp * G, G)], dstv)
        pltpu.async_copy(hs.at[srcv.at[0]], rows_a, sem_a)

        @pl.loop(0, G, step=2)
        def _pair(j):
            pltpu.make_async_copy(hs.at[srcv.at[j]], rows_a, sem_a).wait()
            pltpu.async_copy(hs.at[srcv.at[j + 1]], rows_b, sem_b)
            pltpu.sync_copy(rows_a, acc_sh.at[dstv.at[j]], add=True)
            pltpu.make_async_copy(hs.at[srcv.at[j + 1]], rows_b, sem_b).wait()
            pltpu.async_copy(hs.at[srcv.at[jnp.minimum(j + 2, G - 1)]], rows_a, sem_a)
            pltpu.sync_copy(rows_b, acc_sh.at[dstv.at[j + 1]], add=True)

        pltpu.make_async_copy(hs.at[srcv.at[G - 1]], rows_a, sem_a).wait()

    plsc.subcore_barrier()
    pltpu.sync_copy(acc_sh.at[pl.ds(s * ROWS_PER_TILE, ROWS_PER_TILE)],
                    out.at[c, pl.ds(s * ROWS_PER_TILE, ROWS_PER_TILE)])


_sc_agg = pl.kernel(
    _sc_agg_body,
    out_type=jax.ShapeDtypeStruct((NC, N_PAD, D), jnp.float32),
    mesh=_MESH,
    scratch_types=[
        pltpu.VMEM((G, CH), jnp.int32),
        pltpu.VMEM((G, CH), jnp.int32),
        pltpu.VMEM((CH, D), jnp.float32),
        pltpu.VMEM((CH, D), jnp.float32),
        pltpu.VMEM((ZROWS, D), jnp.float32),
        pltpu.SemaphoreType.DMA,
        pltpu.SemaphoreType.DMA,
        pltpu.VMEM_SHARED((N_PAD, D), jnp.float32),
    ],
)


RB = 400
GRID = N // RB


def _tc1_body(deg_ref, x_ref, w_ref, hs_ref, dis_ref):
    dp = deg_ref[...]
    deg = 1.0 + jnp.sum(dp[0] + dp[1], axis=-1, keepdims=True) * (1.0 / 16.0)
    dis = lax.rsqrt(deg)
    h = jnp.dot(x_ref[...], w_ref[...], preferred_element_type=jnp.float32)
    hs_ref[...] = dis * h
    dis_ref[...] = dis


def _tc2_body(parts_ref, hs_ref, dis_ref, b_ref, w_ref, out_ref):
    p = parts_ref[...]
    dis = dis_ref[...]
    z = dis * (p[0] + p[1] + hs_ref[...]) + b_ref[...]
    h = jnp.maximum(z, 0.0)
    out_ref[...] = dis * jnp.dot(h, w_ref[...], preferred_element_type=jnp.float32)


def _tc3_body(parts_ref, hs_ref, dis_ref, b_ref, out_ref):
    p = parts_ref[...]
    out_ref[...] = dis_ref[...] * (p[0] + p[1] + hs_ref[...]) + b_ref[...]


_tc1 = pl.pallas_call(
    _tc1_body,
    grid=(GRID,),
    in_specs=[
        pl.BlockSpec((NC, RB, 16), lambda i: (0, i, 0)),
        pl.BlockSpec((RB, D), lambda i: (i, 0)),
        pl.BlockSpec((D, D), lambda i: (0, 0)),
    ],
    out_specs=[
        pl.BlockSpec((RB, D), lambda i: (i, 0)),
        pl.BlockSpec((RB, 1), lambda i: (i, 0)),
    ],
    out_shape=[
        jax.ShapeDtypeStruct((N, D), jnp.float32),
        jax.ShapeDtypeStruct((N, 1), jnp.float32),
    ],
)

_tc2 = pl.pallas_call(
    _tc2_body,
    grid=(GRID,),
    in_specs=[
        pl.BlockSpec((NC, RB, D), lambda i: (0, i, 0)),
        pl.BlockSpec((RB, D), lambda i: (i, 0)),
        pl.BlockSpec((RB, 1), lambda i: (i, 0)),
        pl.BlockSpec((1, D), lambda i: (0, 0)),
        pl.BlockSpec((D, D), lambda i: (0, 0)),
    ],
    out_specs=pl.BlockSpec((RB, D), lambda i: (i, 0)),
    out_shape=jax.ShapeDtypeStruct((N, D), jnp.float32),
)

_tc3 = pl.pallas_call(
    _tc3_body,
    grid=(GRID,),
    in_specs=[
        pl.BlockSpec((NC, RB, D), lambda i: (0, i, 0)),
        pl.BlockSpec((RB, D), lambda i: (i, 0)),
        pl.BlockSpec((RB, 1), lambda i: (i, 0)),
        pl.BlockSpec((1, D), lambda i: (0, 0)),
    ],
    out_specs=pl.BlockSpec((RB, D), lambda i: (i, 0)),
    out_shape=jax.ShapeDtypeStruct((N, D), jnp.float32),
)


def kernel(x, edge_index, W1, b1, W2, b2):
    npad = E_PAD - E
    pad_src = jnp.arange(npad, dtype=jnp.int32) % N
    pad_dst = N + (jnp.arange(npad, dtype=jnp.int32) % (N_PAD - N))
    src2d = jnp.concatenate(
        [edge_index[0], pad_src]).reshape(E_PAD // CH, CH)
    dst2d = jnp.concatenate([edge_index[1], pad_dst]).reshape(E_PAD // CH, CH)

    deg_parts = _sc_deg(dst2d)
    hs1, dis = _tc1(deg_parts, x, W1)
    parts1 = _sc_agg(hs1, src2d, dst2d)
    hs2 = _tc2(parts1, hs1, dis, b1.reshape(1, D), W2)
    parts2 = _sc_agg(hs2, src2d, dst2d)
    return _tc3(parts2, hs2, dis, b2.reshape(1, D))
